# baseline (speedup 1.0000x reference)
import jax
import jax.numpy as jnp
from jax.experimental import pallas as pl
from jax.experimental.pallas import tpu as pltpu

N = 2048
F = 128
C1 = 32
C2 = 32
KC = 4
BLK2 = 512
BLKF = 256
FC = 8

_f32 = jnp.float32
_bf16 = jnp.bfloat16


def _s1_kernel(l0_ref, x_ref, p1_ref, b1_ref, y0_ref):
    l0 = l0_ref[...]
    x0 = x_ref[...]
    x0f = x0.astype(_f32)
    x1f = jnp.dot(l0, x0, preferred_element_type=_f32)
    x1 = x1f.astype(_bf16)
    x2f = 2.0 * jnp.dot(l0, x1, preferred_element_type=_f32) - x0f
    x2 = x2f.astype(_bf16)
    x3f = 2.0 * jnp.dot(l0, x2, preferred_element_type=_f32) - x1f
    x3 = x3f.astype(_bf16)
    m = jnp.concatenate([x0, x1, x2, x3], axis=1)
    p1 = p1_ref[...]
    b1 = b1_ref[...]
    for i in range(4):
        blk = m[i * 512:(i + 1) * 512, :]
        o = jnp.dot(blk, p1, preferred_element_type=_f32) + b1
        y0_ref[i * 512:(i + 1) * 512, :] = jnp.maximum(o, 0.0).astype(_bf16)


def _step1_kernel(l_ref, yfull_ref, o_ref):
    z = jnp.dot(l_ref[...], yfull_ref[...], preferred_element_type=_f32)
    o_ref[...] = z.astype(_bf16)


def _step2_kernel(l_ref, yfull_ref, yprev_ref, o_ref):
    z = jnp.dot(l_ref[...], yfull_ref[...], preferred_element_type=_f32)
    o_ref[...] = (2.0 * z - yprev_ref[...].astype(_f32)).astype(_bf16)


def _final_kernel(l_ref, y2full_ref, y0_ref, y1_ref, y2_ref, p2_ref, b2_ref,
                  wfc_ref, o_ref):
    z = jnp.dot(l_ref[...], y2full_ref[...], preferred_element_type=_f32)
    y3 = (2.0 * z - y1_ref[...].astype(_f32)).astype(_bf16)
    y0 = y0_ref[...]
    y1 = y1_ref[...]
    y2 = y2_ref[...]
    p2 = p2_ref[...]
    b2 = b2_ref[...]
    acc = jnp.zeros((o_ref.shape[0], o_ref.shape[1]), _f32)
    w = FC * C1
    for c in range(F // FC):
        sl = slice(c * w, (c + 1) * w)
        cat = jnp.concatenate([y0[:, sl], y1[:, sl], y2[:, sl], y3[:, sl]],
                              axis=1)
        g = jnp.dot(cat, p2, preferred_element_type=_f32) + b2
        g = jnp.maximum(g, 0.0).astype(_bf16)
        acc = acc + jnp.dot(g, wfc_ref[sl, :], preferred_element_type=_f32)
    o_ref[...] = acc


def kernel(x, L, W_g1, b_g1, W_g2, b_g2, W_fc1, b_fc1, W_fc2, b_fc2):
    L0 = L[0].astype(_bf16)
    L2 = L[2].astype(_bf16)
    xb = x.astype(_bf16)

    eyef = jnp.eye(F, dtype=_f32)
    P1 = jnp.einsum('fg,ck->kfgc', eyef, W_g1).reshape(KC * F, F * C1)
    P1 = P1.astype(_bf16)
    b1r = jnp.tile(b_g1, F).reshape(1, F * C1)
    W2km = W_g2.reshape(C2, C1, KC)
    eyec = jnp.eye(FC, dtype=_f32)
    P2 = jnp.einsum('fg,cak->kfagc', eyec, W2km).reshape(KC * FC * C1, FC * C2)
    P2 = P2.astype(_bf16)
    b2r = jnp.tile(b_g2, FC).reshape(1, FC * C2)
    WfcT = (W_fc2 @ W_fc1).T.astype(_bf16)
    bfc = W_fc2 @ b_fc1 + b_fc2

    cp = pltpu.CompilerParams(vmem_limit_bytes=60 * 1024 * 1024)

    y0 = pl.pallas_call(
        _s1_kernel,
        out_shape=jax.ShapeDtypeStruct((N, F * C1), _bf16),
        compiler_params=cp,
    )(L0, xb, P1, b1r)

    spec_l2 = pl.BlockSpec((BLK2, N), lambda i: (i, 0))
    spec_full = pl.BlockSpec((N, F * C1), lambda i: (0, 0))
    spec_blk2 = pl.BlockSpec((BLK2, F * C1), lambda i: (i, 0))

    y1 = pl.pallas_call(
        _step1_kernel,
        grid=(N // BLK2,),
        in_specs=[spec_l2, spec_full],
        out_specs=spec_blk2,
        out_shape=jax.ShapeDtypeStruct((N, F * C1), _bf16),
        compiler_params=cp,
    )(L2, y0)

    y2 = pl.pallas_call(
        _step2_kernel,
        grid=(N // BLK2,),
        in_specs=[spec_l2, spec_full, spec_blk2],
        out_specs=spec_blk2,
        out_shape=jax.ShapeDtypeStruct((N, F * C1), _bf16),
        compiler_params=cp,
    )(L2, y1, y0)

    spec_lf = pl.BlockSpec((BLKF, N), lambda i: (i, 0))
    spec_blkf = pl.BlockSpec((BLKF, F * C1), lambda i: (i, 0))
    out = pl.pallas_call(
        _final_kernel,
        grid=(N // BLKF,),
        in_specs=[
            spec_lf, spec_full, spec_blkf, spec_blkf, spec_blkf,
            pl.BlockSpec((KC * FC * C1, FC * C2), lambda i: (0, 0)),
            pl.BlockSpec((1, FC * C2), lambda i: (0, 0)),
            pl.BlockSpec((F * C2, 10), lambda i: (0, 0)),
        ],
        out_specs=pl.BlockSpec((BLKF, 10), lambda i: (i, 0)),
        out_shape=jax.ShapeDtypeStruct((N, 10), _f32),
        compiler_params=cp,
    )(L2, y2, y0, y1, y2, P2, b2r, WfcT)

    return out + bfc


# speedup vs baseline: 1.1024x; 1.1024x over previous
import jax
import jax.numpy as jnp
from jax.experimental import pallas as pl
from jax.experimental.pallas import tpu as pltpu

N = 2048
F = 128
C1 = 32
C2 = 32
KC = 4
BLK2 = 512
BLKF = 512
FC = 4

_f32 = jnp.float32
_bf16 = jnp.bfloat16


def _s1_kernel(l0_ref, x_ref, p1_ref, b1_ref, y0_ref):
    l0 = l0_ref[...]
    x0 = x_ref[...]
    x0f = x0.astype(_f32)
    x1f = jnp.dot(l0, x0, preferred_element_type=_f32)
    x1 = x1f.astype(_bf16)
    x2f = 2.0 * jnp.dot(l0, x1, preferred_element_type=_f32) - x0f
    x2 = x2f.astype(_bf16)
    x3f = 2.0 * jnp.dot(l0, x2, preferred_element_type=_f32) - x1f
    x3 = x3f.astype(_bf16)
    m = jnp.concatenate([x0, x1, x2, x3], axis=1)
    p1 = p1_ref[...]
    b1 = b1_ref[...]
    for i in range(4):
        blk = m[i * 512:(i + 1) * 512, :]
        o = jnp.dot(blk, p1, preferred_element_type=_f32) + b1
        y0_ref[i * 512:(i + 1) * 512, :] = jnp.maximum(o, 0.0).astype(_bf16)


def _step1_kernel(l_ref, yfull_ref, o_ref):
    z = jnp.dot(l_ref[...], yfull_ref[...], preferred_element_type=_f32)
    o_ref[...] = z.astype(_bf16)


def _step2_kernel(l_ref, yfull_ref, yprev_ref, o_ref):
    z = jnp.dot(l_ref[...], yfull_ref[...], preferred_element_type=_f32)
    o_ref[...] = (2.0 * z - yprev_ref[...].astype(_f32)).astype(_bf16)


def _final_kernel(l_ref, y2full_ref, y0_ref, y1_ref, p2_ref, b2_ref,
                  wfc_ref, o_ref, g_ref):
    z = jnp.dot(l_ref[...], y2full_ref[...], preferred_element_type=_f32)
    y3 = (2.0 * z - y1_ref[...].astype(_f32)).astype(_bf16)
    y0 = y0_ref[...]
    y1 = y1_ref[...]
    row0 = pl.multiple_of(pl.program_id(0) * BLKF, BLKF)
    y2 = y2full_ref[pl.ds(row0, BLKF), :]
    p2 = p2_ref[...]
    b2 = b2_ref[...]
    w = FC * C1
    for c in range(F // FC):
        sl = slice(c * w, (c + 1) * w)
        cat = jnp.concatenate([y0[:, sl], y1[:, sl], y2[:, sl], y3[:, sl]],
                              axis=1)
        g = jnp.dot(cat, p2, preferred_element_type=_f32) + b2
        g_ref[:, sl] = jnp.maximum(g, 0.0).astype(_bf16)
    o_ref[...] = jnp.dot(g_ref[...], wfc_ref[...],
                         preferred_element_type=_f32)


def kernel(x, L, W_g1, b_g1, W_g2, b_g2, W_fc1, b_fc1, W_fc2, b_fc2):
    L0 = L[0].astype(_bf16)
    L2 = L[2].astype(_bf16)
    xb = x.astype(_bf16)

    eyef = jnp.eye(F, dtype=_f32)
    P1 = jnp.einsum('fg,ck->kfgc', eyef, W_g1).reshape(KC * F, F * C1)
    P1 = P1.astype(_bf16)
    b1r = jnp.tile(b_g1, F).reshape(1, F * C1)
    W2km = W_g2.reshape(C2, C1, KC)
    eyec = jnp.eye(FC, dtype=_f32)
    P2 = jnp.einsum('fg,cak->kfagc', eyec, W2km).reshape(KC * FC * C1, FC * C2)
    P2 = P2.astype(_bf16)
    b2r = jnp.tile(b_g2, FC).reshape(1, FC * C2)
    WfcT = (W_fc2 @ W_fc1).T.astype(_bf16)
    bfc = W_fc2 @ b_fc1 + b_fc2

    cp = pltpu.CompilerParams(vmem_limit_bytes=60 * 1024 * 1024)

    y0 = pl.pallas_call(
        _s1_kernel,
        out_shape=jax.ShapeDtypeStruct((N, F * C1), _bf16),
        compiler_params=cp,
    )(L0, xb, P1, b1r)

    spec_l2 = pl.BlockSpec((BLK2, N), lambda i: (i, 0))
    spec_full = pl.BlockSpec((N, F * C1), lambda i: (0, 0))
    spec_blk2 = pl.BlockSpec((BLK2, F * C1), lambda i: (i, 0))

    y1 = pl.pallas_call(
        _step1_kernel,
        grid=(N // BLK2,),
        in_specs=[spec_l2, spec_full],
        out_specs=spec_blk2,
        out_shape=jax.ShapeDtypeStruct((N, F * C1), _bf16),
        compiler_params=cp,
    )(L2, y0)

    y2 = pl.pallas_call(
        _step2_kernel,
        grid=(N // BLK2,),
        in_specs=[spec_l2, spec_full, spec_blk2],
        out_specs=spec_blk2,
        out_shape=jax.ShapeDtypeStruct((N, F * C1), _bf16),
        compiler_params=cp,
    )(L2, y1, y0)

    spec_lf = pl.BlockSpec((BLKF, N), lambda i: (i, 0))
    spec_blkf = pl.BlockSpec((BLKF, F * C1), lambda i: (i, 0))
    out = pl.pallas_call(
        _final_kernel,
        grid=(N // BLKF,),
        in_specs=[
            spec_lf, spec_full, spec_blkf, spec_blkf,
            pl.BlockSpec((KC * FC * C1, FC * C2), lambda i: (0, 0)),
            pl.BlockSpec((1, FC * C2), lambda i: (0, 0)),
            pl.BlockSpec((F * C2, 10), lambda i: (0, 0)),
        ],
        out_specs=pl.BlockSpec((BLKF, 10), lambda i: (i, 0)),
        out_shape=jax.ShapeDtypeStruct((N, 10), _f32),
        scratch_shapes=[pltpu.VMEM((BLKF, F * C2), _bf16)],
        compiler_params=cp,
    )(L2, y2, y0, y1, P2, b2r, WfcT)

    return out + bfc


# speedup vs baseline: 1.2065x; 1.0945x over previous
import jax
import jax.numpy as jnp
from jax.experimental import pallas as pl
from jax.experimental.pallas import tpu as pltpu

N = 2048
F = 128
C1 = 32
C2 = 32
KC = 4
BLK = 256
FC = 4

_f32 = jnp.float32
_bf16 = jnp.bfloat16


def _s1_kernel(l0_ref, x_ref, p1_ref, b1_ref, y0_ref):
    l0 = l0_ref[...]
    x0 = x_ref[...]
    x0f = x0.astype(_f32)
    x1f = jnp.dot(l0, x0, preferred_element_type=_f32)
    x1 = x1f.astype(_bf16)
    x2f = 2.0 * jnp.dot(l0, x1, preferred_element_type=_f32) - x0f
    x2 = x2f.astype(_bf16)
    x3f = 2.0 * jnp.dot(l0, x2, preferred_element_type=_f32) - x1f
    x3 = x3f.astype(_bf16)
    m = jnp.concatenate([x0, x1, x2, x3], axis=1)
    p1 = p1_ref[...]
    b1 = b1_ref[...]
    for i in range(4):
        blk = m[i * 512:(i + 1) * 512, :]
        o = jnp.dot(blk, p1, preferred_element_type=_f32) + b1
        y0_ref[i * 512:(i + 1) * 512, :] = jnp.maximum(o, 0.0).astype(_bf16)


def _s2_kernel(l_ref, y0full_ref, p2_ref, b2_ref, wfc_ref, o_ref,
               y1s_ref, y2s_ref, g_ref):
    p = pl.program_id(0)
    i = pl.program_id(1)
    row0 = pl.multiple_of(i * BLK, BLK)

    @pl.when(p == 0)
    def _phase0():
        z = jnp.dot(l_ref[...], y0full_ref[...], preferred_element_type=_f32)
        y1s_ref[pl.ds(row0, BLK), :] = z.astype(_bf16)
        o_ref[...] = jnp.zeros_like(o_ref)

    @pl.when(p == 1)
    def _phase1():
        z = jnp.dot(l_ref[...], y1s_ref[...], preferred_element_type=_f32)
        y0b = y0full_ref[pl.ds(row0, BLK), :]
        y2s_ref[pl.ds(row0, BLK), :] = (2.0 * z - y0b.astype(_f32)).astype(_bf16)
        o_ref[...] = jnp.zeros_like(o_ref)

    @pl.when(p == 2)
    def _phase2():
        z = jnp.dot(l_ref[...], y2s_ref[...], preferred_element_type=_f32)
        y1 = y1s_ref[pl.ds(row0, BLK), :]
        y3 = (2.0 * z - y1.astype(_f32)).astype(_bf16)
        y0 = y0full_ref[pl.ds(row0, BLK), :]
        y2 = y2s_ref[pl.ds(row0, BLK), :]
        p2 = p2_ref[...]
        b2 = b2_ref[...]
        w = FC * C1
        for c in range(F // FC):
            sl = slice(c * w, (c + 1) * w)
            cat = jnp.concatenate([y0[:, sl], y1[:, sl], y2[:, sl],
                                   y3[:, sl]], axis=1)
            g = jnp.dot(cat, p2, preferred_element_type=_f32) + b2
            g_ref[:, sl] = jnp.maximum(g, 0.0).astype(_bf16)
        o_ref[...] = jnp.dot(g_ref[...], wfc_ref[...],
                             preferred_element_type=_f32)


def kernel(x, L, W_g1, b_g1, W_g2, b_g2, W_fc1, b_fc1, W_fc2, b_fc2):
    L0 = L[0].astype(_bf16)
    L2 = L[2].astype(_bf16)
    xb = x.astype(_bf16)

    eyef = jnp.eye(F, dtype=_f32)
    P1 = jnp.einsum('fg,ck->kfgc', eyef, W_g1).reshape(KC * F, F * C1)
    P1 = P1.astype(_bf16)
    b1r = jnp.tile(b_g1, F).reshape(1, F * C1)
    W2km = W_g2.reshape(C2, C1, KC)
    eyec = jnp.eye(FC, dtype=_f32)
    P2 = jnp.einsum('fg,cak->kfagc', eyec, W2km).reshape(KC * FC * C1, FC * C2)
    P2 = P2.astype(_bf16)
    b2r = jnp.tile(b_g2, FC).reshape(1, FC * C2)
    WfcT = (W_fc2 @ W_fc1).T.astype(_bf16)
    bfc = W_fc2 @ b_fc1 + b_fc2

    cp = pltpu.CompilerParams(vmem_limit_bytes=62 * 1024 * 1024)

    y0 = pl.pallas_call(
        _s1_kernel,
        out_shape=jax.ShapeDtypeStruct((N, F * C1), _bf16),
        compiler_params=cp,
    )(L0, xb, P1, b1r)

    out = pl.pallas_call(
        _s2_kernel,
        grid=(3, N // BLK),
        in_specs=[
            pl.BlockSpec((BLK, N), lambda p, i: (i, 0)),
            pl.BlockSpec((N, F * C1), lambda p, i: (0, 0)),
            pl.BlockSpec((KC * FC * C1, FC * C2), lambda p, i: (0, 0)),
            pl.BlockSpec((1, FC * C2), lambda p, i: (0, 0)),
            pl.BlockSpec((F * C2, 10), lambda p, i: (0, 0)),
        ],
        out_specs=pl.BlockSpec((BLK, 10), lambda p, i: (i, 0)),
        out_shape=jax.ShapeDtypeStruct((N, 10), _f32),
        scratch_shapes=[
            pltpu.VMEM((N, F * C1), _bf16),
            pltpu.VMEM((N, F * C1), _bf16),
            pltpu.VMEM((BLK, F * C2), _bf16),
        ],
        compiler_params=cp,
    )(L2, y0, P2, b2r, WfcT)

    return out + bfc


# speedup vs baseline: 1.3127x; 1.0879x over previous
import jax
import jax.numpy as jnp
from jax.experimental import pallas as pl
from jax.experimental.pallas import tpu as pltpu

N = 2048
F = 128
C1 = 32
C2 = 32
KC = 4
BLK = 256
FC = 4

_f32 = jnp.float32
_bf16 = jnp.bfloat16


def _s1_kernel(l0_ref, x_ref, p1_ref, b1_ref, y0_ref, m_ref):
    s = pl.program_id(0)

    @pl.when(s == 0)
    def _recursion():
        l0 = l0_ref[...].astype(_bf16)
        x0 = x_ref[...].astype(_bf16)
        x0f = x_ref[...]
        x1f = jnp.dot(l0, x0, preferred_element_type=_f32)
        x1 = x1f.astype(_bf16)
        x2f = 2.0 * jnp.dot(l0, x1, preferred_element_type=_f32) - x0f
        x2 = x2f.astype(_bf16)
        x3f = 2.0 * jnp.dot(l0, x2, preferred_element_type=_f32) - x1f
        x3 = x3f.astype(_bf16)
        m_ref[...] = jnp.concatenate([x0, x1, x2, x3], axis=1)

    @pl.when(s > 0)
    def _combine():
        row0 = pl.multiple_of((s - 1) * BLK, BLK)
        blk = m_ref[pl.ds(row0, BLK), :]
        o = jnp.dot(blk, p1_ref[...], preferred_element_type=_f32) + b1_ref[...]
        y0_ref[...] = jnp.maximum(o, 0.0).astype(_bf16)


def _s2_kernel(l_ref, y0full_ref, p2_ref, b2_ref, wfc_ref, bfc_ref, o_ref,
               y1s_ref, y2s_ref, g_ref):
    p = pl.program_id(0)
    i = pl.program_id(1)
    row0 = pl.multiple_of(i * BLK, BLK)
    l = l_ref[...].astype(_bf16)

    @pl.when(p == 0)
    def _phase0():
        z = jnp.dot(l, y0full_ref[...], preferred_element_type=_f32)
        y1s_ref[pl.ds(row0, BLK), :] = z.astype(_bf16)
        o_ref[...] = jnp.zeros_like(o_ref)

    @pl.when(p == 1)
    def _phase1():
        z = jnp.dot(l, y1s_ref[...], preferred_element_type=_f32)
        y0b = y0full_ref[pl.ds(row0, BLK), :]
        y2s_ref[pl.ds(row0, BLK), :] = (2.0 * z - y0b.astype(_f32)).astype(_bf16)
        o_ref[...] = jnp.zeros_like(o_ref)

    @pl.when(p == 2)
    def _phase2():
        z = jnp.dot(l, y2s_ref[...], preferred_element_type=_f32)
        y1 = y1s_ref[pl.ds(row0, BLK), :]
        y3 = (2.0 * z - y1.astype(_f32)).astype(_bf16)
        y0 = y0full_ref[pl.ds(row0, BLK), :]
        y2 = y2s_ref[pl.ds(row0, BLK), :]
        p2 = p2_ref[...]
        b2 = b2_ref[...]
        w = FC * C1
        for c in range(F // FC):
            sl = slice(c * w, (c + 1) * w)
            cat = jnp.concatenate([y0[:, sl], y1[:, sl], y2[:, sl],
                                   y3[:, sl]], axis=1)
            g = jnp.dot(cat, p2, preferred_element_type=_f32) + b2
            g_ref[:, sl] = jnp.maximum(g, 0.0).astype(_bf16)
        o_ref[...] = (jnp.dot(g_ref[...], wfc_ref[...],
                              preferred_element_type=_f32)
                      + bfc_ref[0:1, :])


def kernel(x, L, W_g1, b_g1, W_g2, b_g2, W_fc1, b_fc1, W_fc2, b_fc2):
    eyef = jnp.eye(F, dtype=_f32)
    P1 = jnp.einsum('fg,ck->kfgc', eyef, W_g1).reshape(KC * F, F * C1)
    P1 = P1.astype(_bf16)
    b1r = jnp.tile(b_g1, F).reshape(1, F * C1)
    W2km = W_g2.reshape(C2, C1, KC)
    eyec = jnp.eye(FC, dtype=_f32)
    P2 = jnp.einsum('fg,cak->kfagc', eyec, W2km).reshape(KC * FC * C1, FC * C2)
    P2 = P2.astype(_bf16)
    b2r = jnp.tile(b_g2, FC).reshape(1, FC * C2)
    WfcT = (W_fc2 @ W_fc1).T.astype(_bf16)
    bfc = (W_fc2 @ b_fc1 + b_fc2).reshape(1, 10)
    bfcr = jnp.tile(bfc, (8, 1))

    cp = pltpu.CompilerParams(vmem_limit_bytes=62 * 1024 * 1024)

    y0 = pl.pallas_call(
        _s1_kernel,
        grid=(1 + N // BLK,),
        in_specs=[
            pl.BlockSpec((N, N), lambda s: (0, 0)),
            pl.BlockSpec((N, F), lambda s: (0, 0)),
            pl.BlockSpec((KC * F, F * C1), lambda s: (0, 0)),
            pl.BlockSpec((1, F * C1), lambda s: (0, 0)),
        ],
        out_specs=pl.BlockSpec((BLK, F * C1),
                               lambda s: (jnp.maximum(s - 1, 0), 0)),
        out_shape=jax.ShapeDtypeStruct((N, F * C1), _bf16),
        scratch_shapes=[pltpu.VMEM((N, KC * F), _bf16)],
        compiler_params=cp,
    )(L[0], x, P1, b1r)

    out = pl.pallas_call(
        _s2_kernel,
        grid=(3, N // BLK),
        in_specs=[
            pl.BlockSpec((BLK, N), lambda p, i: (i, 0)),
            pl.BlockSpec((N, F * C1), lambda p, i: (0, 0)),
            pl.BlockSpec((KC * FC * C1, FC * C2), lambda p, i: (0, 0)),
            pl.BlockSpec((1, FC * C2), lambda p, i: (0, 0)),
            pl.BlockSpec((F * C2, 10), lambda p, i: (0, 0)),
            pl.BlockSpec((8, 10), lambda p, i: (0, 0)),
        ],
        out_specs=pl.BlockSpec((BLK, 10), lambda p, i: (i, 0)),
        out_shape=jax.ShapeDtypeStruct((N, 10), _f32),
        scratch_shapes=[
            pltpu.VMEM((N, F * C1), _bf16),
            pltpu.VMEM((N, F * C1), _bf16),
            pltpu.VMEM((BLK, F * C2), _bf16),
        ],
        compiler_params=cp,
    )(L[2], y0, P2, b2r, WfcT, bfcr)

    return out


# speedup vs baseline: 1.3174x; 1.0036x over previous
import jax
import jax.numpy as jnp
from jax.experimental import pallas as pl
from jax.experimental.pallas import tpu as pltpu

N = 2048
F = 128
C1 = 32
C2 = 32
KC = 4
BLK = 256
FC = 4

_f32 = jnp.float32
_bf16 = jnp.bfloat16


def _s1_kernel(l0_ref, x_ref, p1_ref, b1_ref, y0_ref, m_ref):
    s = pl.program_id(0)

    @pl.when(s == 0)
    def _recursion():
        l0 = l0_ref[...].astype(_bf16)
        x0 = x_ref[...].astype(_bf16)
        x0f = x_ref[...]
        x1f = jnp.dot(l0, x0, preferred_element_type=_f32)
        x1 = x1f.astype(_bf16)
        x2f = 2.0 * jnp.dot(l0, x1, preferred_element_type=_f32) - x0f
        x2 = x2f.astype(_bf16)
        x3f = 2.0 * jnp.dot(l0, x2, preferred_element_type=_f32) - x1f
        x3 = x3f.astype(_bf16)
        m_ref[...] = jnp.concatenate([x0, x1, x2, x3], axis=1)

    @pl.when(s > 0)
    def _combine():
        row0 = pl.multiple_of((s - 1) * BLK, BLK)
        blk = m_ref[pl.ds(row0, BLK), :]
        o = jnp.dot(blk, p1_ref[...], preferred_element_type=_f32)
        ob = o.astype(_bf16) + b1_ref[...]
        y0_ref[...] = jnp.maximum(ob, jnp.bfloat16(0))


def _s2_kernel(l_ref, y0full_ref, p2_ref, b2_ref, wfc_ref, bfc_ref, o_ref,
               y1s_ref, y2s_ref, g_ref):
    p = pl.program_id(0)
    i = pl.program_id(1)
    row0 = pl.multiple_of(i * BLK, BLK)
    l = l_ref[...].astype(_bf16)

    @pl.when(p == 0)
    def _phase0():
        z = jnp.dot(l, y0full_ref[...], preferred_element_type=_f32)
        y1s_ref[pl.ds(row0, BLK), :] = z.astype(_bf16)
        o_ref[...] = jnp.zeros_like(o_ref)

    @pl.when(p == 1)
    def _phase1():
        z = jnp.dot(l, y1s_ref[...], preferred_element_type=_f32)
        y0b = y0full_ref[pl.ds(row0, BLK), :]
        zb = z.astype(_bf16)
        y2s_ref[pl.ds(row0, BLK), :] = jnp.bfloat16(2) * zb - y0b
        o_ref[...] = jnp.zeros_like(o_ref)

    @pl.when(p == 2)
    def _phase2():
        z = jnp.dot(l, y2s_ref[...], preferred_element_type=_f32)
        y1 = y1s_ref[pl.ds(row0, BLK), :]
        y3 = jnp.bfloat16(2) * z.astype(_bf16) - y1
        y0 = y0full_ref[pl.ds(row0, BLK), :]
        y2 = y2s_ref[pl.ds(row0, BLK), :]
        p2 = p2_ref[...]
        b2 = b2_ref[...]
        w = FC * C1
        for c in range(F // FC):
            sl = slice(c * w, (c + 1) * w)
            cat = jnp.concatenate([y0[:, sl], y1[:, sl], y2[:, sl],
                                   y3[:, sl]], axis=1)
            g = jnp.dot(cat, p2, preferred_element_type=_f32)
            gb = g.astype(_bf16) + b2
            g_ref[:, sl] = jnp.maximum(gb, jnp.bfloat16(0))
        o_ref[...] = (jnp.dot(g_ref[...], wfc_ref[...],
                              preferred_element_type=_f32)
                      + bfc_ref[0:1, :])


def kernel(x, L, W_g1, b_g1, W_g2, b_g2, W_fc1, b_fc1, W_fc2, b_fc2):
    eyef = jnp.eye(F, dtype=_bf16)
    P1 = jnp.einsum('fg,ck->kfgc', eyef, W_g1.astype(_bf16))
    P1 = P1.reshape(KC * F, F * C1)
    b1r = jnp.tile(b_g1, F).reshape(1, F * C1).astype(_bf16)
    W2km = W_g2.astype(_bf16).reshape(C2, C1, KC)
    eyec = jnp.eye(FC, dtype=_bf16)
    P2 = jnp.einsum('fg,cak->kfagc', eyec, W2km).reshape(KC * FC * C1, FC * C2)
    b2r = jnp.tile(b_g2, FC).reshape(1, FC * C2).astype(_bf16)
    WfcT = (W_fc2 @ W_fc1).T.astype(_bf16)
    bfc = (W_fc2 @ b_fc1 + b_fc2).reshape(1, 10)
    bfcr = jnp.tile(bfc, (8, 1))

    cp = pltpu.CompilerParams(vmem_limit_bytes=62 * 1024 * 1024)

    y0 = pl.pallas_call(
        _s1_kernel,
        grid=(1 + N // BLK,),
        in_specs=[
            pl.BlockSpec((N, N), lambda s: (0, 0)),
            pl.BlockSpec((N, F), lambda s: (0, 0)),
            pl.BlockSpec((KC * F, F * C1), lambda s: (0, 0)),
            pl.BlockSpec((1, F * C1), lambda s: (0, 0)),
        ],
        out_specs=pl.BlockSpec((BLK, F * C1),
                               lambda s: (jnp.maximum(s - 1, 0), 0)),
        out_shape=jax.ShapeDtypeStruct((N, F * C1), _bf16),
        scratch_shapes=[pltpu.VMEM((N, KC * F), _bf16)],
        compiler_params=cp,
    )(L[0], x, P1, b1r)

    out = pl.pallas_call(
        _s2_kernel,
        grid=(3, N // BLK),
        in_specs=[
            pl.BlockSpec((BLK, N), lambda p, i: (i, 0)),
            pl.BlockSpec((N, F * C1), lambda p, i: (0, 0)),
            pl.BlockSpec((KC * FC * C1, FC * C2), lambda p, i: (0, 0)),
            pl.BlockSpec((1, FC * C2), lambda p, i: (0, 0)),
            pl.BlockSpec((F * C2, 10), lambda p, i: (0, 0)),
            pl.BlockSpec((8, 10), lambda p, i: (0, 0)),
        ],
        out_specs=pl.BlockSpec((BLK, 10), lambda p, i: (i, 0)),
        out_shape=jax.ShapeDtypeStruct((N, 10), _f32),
        scratch_shapes=[
            pltpu.VMEM((N, F * C1), _bf16),
            pltpu.VMEM((N, F * C1), _bf16),
            pltpu.VMEM((BLK, F * C2), _bf16),
        ],
        compiler_params=cp,
    )(L[2], y0, P2, b2r, WfcT, bfcr)

    return out
